# baseline (speedup 1.0000x reference)
import functools

import jax
import jax.numpy as jnp
from jax import lax
from jax.experimental import pallas as pl
from jax.experimental.pallas import tpu as pltpu
from jax.experimental.pallas import tpu_sc as plsc

V = 10000
H = 128
E = 160000
NUM_POLICY = V * (V - 1) // 2

NC = 2
NS = 16
NWK = NC * NS
L = 16

VP = 10240
SLAB = VP // NS
EP = 163840
EPW = EP // NWK
GW = EPW // 128
EPR = EP // 128
BLK = 2048
EB = EP // BLK
SB = 10
FILL_BLK = 1048576

_sc_mesh = functools.partial(
    plsc.VectorSubcoreMesh, core_axis_name="c", subcore_axis_name="s")


def _wid():
    return lax.axis_index("c") * NS + lax.axis_index("s")


def _sc_degree_body(dst2d, cnt, acc, zbuf, ones, idxv, sem):
    c = lax.axis_index("c")
    s = lax.axis_index("s")
    w = _wid()

    def zero16(k, _):
        zbuf[pl.ds(k * L, L)] = jnp.zeros((L,), jnp.float32)
        return _
    lax.fori_loop(0, SLAB // L, zero16, None)

    def one16(k, _):
        ones[pl.ds(k * L, L)] = jnp.full((L,), 1.0, jnp.float32)
        return _
    lax.fori_loop(0, 128 // L, one16, None)

    pltpu.sync_copy(zbuf, acc.at[pl.ds(s * SLAB, SLAB)])
    plsc.subcore_barrier()

    pltpu.sync_copy(dst2d.at[pl.ds(w * GW, GW)], idxv)

    def body(j, _):
        pltpu.sync_copy(ones, acc.at[idxv.at[j]], add=True)
        return _
    lax.fori_loop(0, GW, body, None)

    plsc.subcore_barrier()
    pltpu.sync_copy(acc.at[pl.ds(s * SLAB, SLAB)],
                    cnt.at[c, pl.ds(s * SLAB, SLAB)])


def _sc_degree(dst2d):
    return pl.kernel(
        _sc_degree_body,
        out_type=jax.ShapeDtypeStruct((NC, VP), jnp.float32),
        mesh=_sc_mesh(),
        scratch_types=[
            pltpu.VMEM_SHARED((VP,), jnp.float32),
            pltpu.VMEM((SLAB,), jnp.float32),
            pltpu.VMEM((128,), jnp.float32),
            pltpu.VMEM((GW, 128), jnp.int32),
            pltpu.SemaphoreType.DMA,
        ],
    )(dst2d)


GSZ = 64
GW2 = EPW // GSZ


def _sc_layer_body(tabS, tabD, hd, src2d, dst2d, gout, parts,
                   acc, idxs, idxd, bufA, bufB, bufC, semA, semB, semC):
    c = lax.axis_index("c")
    s = lax.axis_index("s")
    w = _wid()

    def zero16(t, _):
        row = t // 8
        col = (t % 8) * L
        bufC[row, pl.ds(col, L)] = jnp.zeros((L,), jnp.float32)
        return _
    lax.fori_loop(0, GSZ * 8, zero16, None)

    def zslab(k, _):
        pltpu.sync_copy(bufC, acc.at[pl.ds(s * SLAB + k * GSZ, GSZ)])
        return _
    lax.fori_loop(0, SLAB // GSZ, zslab, None)
    plsc.subcore_barrier()

    pltpu.sync_copy(src2d.at[pl.ds(w * GW2, GW2)], idxs)
    pltpu.sync_copy(dst2d.at[pl.ds(w * GW2, GW2)], idxd)

    def body(j, _):
        da = pltpu.async_copy(tabS.at[idxs.at[j]], bufA, semA)
        db = pltpu.async_copy(tabD.at[idxd.at[j]], bufB, semB)
        dc = pltpu.async_copy(hd.at[idxs.at[j]], bufC, semC)
        da.wait()
        db.wait()

        @plsc.parallel_loop(0, GSZ * 8, 1, unroll=8)
        def add16(t):
            row = t // 8
            col = (t % 8) * L
            bufA[row, pl.ds(col, L)] = (bufA[row, pl.ds(col, L)]
                                        + bufB[row, pl.ds(col, L)])

        pltpu.sync_copy(bufA, gout.at[pl.ds(w * EPW + j * GSZ, GSZ)])
        dc.wait()
        pltpu.sync_copy(bufC, acc.at[idxd.at[j]], add=True)
        return _
    lax.fori_loop(0, GW2, body, None)

    plsc.subcore_barrier()
    pltpu.sync_copy(acc.at[pl.ds(s * SLAB, SLAB)],
                    parts.at[c, pl.ds(s * SLAB, SLAB)])


def _sc_layer(tabS, tabD, hd, src64, dst64):
    return pl.kernel(
        _sc_layer_body,
        out_type=(jax.ShapeDtypeStruct((EP, H), jnp.float32),
                  jax.ShapeDtypeStruct((NC, VP, H), jnp.float32)),
        mesh=_sc_mesh(),
        scratch_types=[
            pltpu.VMEM_SHARED((VP, H), jnp.float32),
            pltpu.VMEM((GW2, GSZ), jnp.int32),
            pltpu.VMEM((GW2, GSZ), jnp.int32),
            pltpu.VMEM((GSZ, H), jnp.float32),
            pltpu.VMEM((GSZ, H), jnp.float32),
            pltpu.VMEM((GSZ, H), jnp.float32),
            pltpu.SemaphoreType.DMA,
            pltpu.SemaphoreType.DMA,
            pltpu.SemaphoreType.DMA,
        ],
    )(tabS, tabD, hd, src64, dst64)


def _sc_policy_scatter_body(idx2d, val2d, pol, idxv, valv, sem):
    w = _wid()
    pltpu.sync_copy(idx2d.at[pl.ds(w * GW, GW)], idxv)
    pltpu.sync_copy(val2d.at[pl.ds(w * GW, GW)], valv)

    descs = [pltpu.async_copy(valv.at[j], pol.at[idxv.at[j]], sem)
             for j in range(GW)]
    for d in descs:
        d.wait()


def _sc_policy_scatter(pol_ref, idx2d, val2d):
    pl.kernel(
        _sc_policy_scatter_body,
        out_type=(),
        mesh=_sc_mesh(),
        scratch_types=[
            pltpu.VMEM((GW, 128), jnp.int32),
            pltpu.VMEM((GW, 128), jnp.float32),
            pltpu.SemaphoreType.DMA,
        ],
    )(idx2d, val2d, pol_ref)


def _full(a):
    nd = a.ndim
    return pl.BlockSpec(a.shape, lambda *_, _nd=nd: (0,) * _nd)


def _n_body(ei_ref, n_ref):
    i = pl.program_id(0)

    @pl.when(i == 0)
    def _():
        n_ref[0, 0] = 0
    m = jnp.max(ei_ref[...])
    n_ref[0, 0] = jnp.maximum(n_ref[0, 0], m + 1)


def _tc_n(ei_flat2d):
    return pl.pallas_call(
        _n_body,
        grid=(1,),
        in_specs=[pl.BlockSpec((2500, 128), lambda i: (0, 0))],
        out_specs=pl.BlockSpec(memory_space=pltpu.SMEM),
        out_shape=jax.ShapeDtypeStruct((1, 1), jnp.int32),
    )(ei_flat2d)


def _prep1_body(n_ref, emb, g1W, e1npW, e1cbW, eeW, eeb, e1epW, e1epb,
                e1npb, e1cbb, e2epW, e2epb, e2npb, e2cbW, e2cbb,
                S1, D1, h1, T1f, c1o, M2o, c2o):
    nn = n_ref[0, 0]
    rows = lax.broadcasted_iota(jnp.int32, (V, 1), 0)
    x0 = jnp.where(rows < nn, emb[...], 0.0)
    dot = functools.partial(jnp.dot, preferred_element_type=jnp.float32)
    pad = jnp.zeros((VP - V, H), jnp.float32)

    S1[pl.ds(0, V), :] = dot(x0, dot(e1npW[0:H, :], e1cbW[0:H, :]))
    S1[pl.ds(V, VP - V), :] = pad
    D1[pl.ds(0, V), :] = dot(x0, dot(e1npW[H:2 * H, :], e1cbW[0:H, :]))
    D1[pl.ds(V, VP - V), :] = pad
    h1[pl.ds(0, V), :] = dot(x0, g1W[...])
    h1[pl.ds(V, VP - V), :] = pad

    cbb = e1cbW[H:2 * H, :]
    T1f[...] = dot(eeW[...], dot(e1epW[...], cbb))
    c1o[...] = (dot(e1npb[...].reshape(1, H), e1cbW[0:H, :])
                + dot((dot(eeb[...].reshape(1, H), e1epW[...])
                       + e1epb[...].reshape(1, H)), cbb)
                + e1cbb[...].reshape(1, H))
    M2o[...] = dot(e2epW[...], e2cbW[H:2 * H, :])
    c2o[...] = (dot(e2npb[...].reshape(1, H), e2cbW[0:H, :])
                + dot(e2epb[...].reshape(1, H), e2cbW[H:2 * H, :])
                + e2cbb[...].reshape(1, H))


def _tc_prep1(n, emb, g1W, e1npW, e1cbW, eeW, eeb, e1epW, e1epb, e1npb,
              e1cbb, e2epW, e2epb, e2npb, e2cbW, e2cbb):
    return pl.pallas_call(
        _prep1_body,
        in_specs=[pl.BlockSpec(memory_space=pltpu.SMEM)] + [
            _full(a)
            for a in (emb, g1W, e1npW, e1cbW, eeW, eeb, e1epW, e1epb,
                      e1npb, e1cbb, e2epW, e2epb, e2npb, e2cbW, e2cbb)],
        out_shape=(
            jax.ShapeDtypeStruct((VP, H), jnp.float32),
            jax.ShapeDtypeStruct((VP, H), jnp.float32),
            jax.ShapeDtypeStruct((VP, H), jnp.float32),
            jax.ShapeDtypeStruct((3, H), jnp.float32),
            jax.ShapeDtypeStruct((1, H), jnp.float32),
            jax.ShapeDtypeStruct((H, H), jnp.float32),
            jax.ShapeDtypeStruct((1, H), jnp.float32),
        ),
    )(n, emb, g1W, e1npW, e1cbW, eeW, eeb, e1epW, e1epb, e1npb, e1cbb,
      e2epW, e2epb, e2npb, e2cbW, e2cbb)


def _dis_body(cnt, h1, dis, hd1):
    c = cnt[0, :] + cnt[1, :] + 1.0
    d = lax.rsqrt(c).reshape(VP, 1)
    dis[...] = d
    hd1[...] = h1[...] * d


def _tc_dis(cnt, h1):
    return pl.pallas_call(
        _dis_body,
        out_shape=(jax.ShapeDtypeStruct((VP, 1), jnp.float32),
                   jax.ShapeDtypeStruct((VP, H), jnp.float32)),
    )(cnt, h1)


def _node_finish(parts, hd, dis, b, bng, bnb, nn):
    acc = parts[0] + parts[1]
    out = dis * (acc + hd) + b[None, :]
    rows = lax.broadcasted_iota(jnp.int32, (VP, 1), 0)
    mask = rows < nn
    nf = nn.astype(jnp.float32)
    m = jnp.sum(jnp.where(mask, out, 0.0), axis=0, keepdims=True) / nf
    cen = jnp.where(mask, out - m, 0.0)
    var = jnp.sum(cen * cen, axis=0, keepdims=True) / nf
    y = bng[None, :] * (out - m) * lax.rsqrt(var + 1e-5) + bnb[None, :]
    return jnp.where(mask, jnp.maximum(y, 0.0), 0.0)


def _layer2_body(n_ref, parts1, hd1, dis, g1b, bn1g, bn1b, g2W, e2npW,
                 e2cbW, S2, D2, hd2):
    nn = n_ref[0, 0]
    x1 = _node_finish(parts1[...], hd1[...], dis[...], g1b[...],
                      bn1g[...], bn1b[...], nn)
    dot = functools.partial(jnp.dot, preferred_element_type=jnp.float32)
    S2[...] = dot(x1, dot(e2npW[0:H, :], e2cbW[0:H, :]))
    D2[...] = dot(x1, dot(e2npW[H:2 * H, :], e2cbW[0:H, :]))
    hd2[...] = dot(x1, g2W[...]) * dis[...]


def _tc_layer2(n, parts1, hd1, dis, g1b, bn1g, bn1b, g2W, e2npW, e2cbW):
    return pl.pallas_call(
        _layer2_body,
        in_specs=[pl.BlockSpec(memory_space=pltpu.SMEM)] + [
            _full(a)
            for a in (parts1, hd1, dis, g1b, bn1g, bn1b, g2W, e2npW,
                      e2cbW)],
        out_shape=(jax.ShapeDtypeStruct((VP, H), jnp.float32),
                   jax.ShapeDtypeStruct((VP, H), jnp.float32),
                   jax.ShapeDtypeStruct((VP, H), jnp.float32)),
    )(n, parts1, hd1, dis, g1b, bn1g, bn1b, g2W, e2npW, e2cbW)


def _value_body(n_ref, parts2, hd2, dis, g2b, bn2g, bn2b, v1W, v1b, v2W,
                v2b, val):
    nn = n_ref[0, 0]
    x2 = _node_finish(parts2[...], hd2[...], dis[...], g2b[...],
                      bn2g[...], bn2b[...], nn)
    nf = nn.astype(jnp.float32)
    xmean = jnp.sum(x2, axis=0, keepdims=True) / nf
    dot = functools.partial(jnp.dot, preferred_element_type=jnp.float32)
    t = jnp.maximum(dot(xmean, v1W[...]) + v1b[...][None, :], 0.0)
    val[...] = jnp.tanh(dot(t, v2W[...]) + v2b[...][None, :])


def _tc_value(n, parts2, hd2, dis, g2b, bn2g, bn2b, v1W, v1b, v2W, v2b):
    return pl.pallas_call(
        _value_body,
        in_specs=[pl.BlockSpec(memory_space=pltpu.SMEM)] + [
            _full(a)
            for a in (parts2, hd2, dis, g2b, bn2g, bn2b, v1W, v1b, v2W,
                      v2b)],
        out_shape=jax.ShapeDtypeStruct((1, 1), jnp.float32),
    )(n, parts2, hd2, dis, g2b, bn2g, bn2b, v1W, v1b, v2W, v2b)


def _edge_z1(g1_blk, eat_blk, T1f, c1):
    ea = lax.dot_general(eat_blk, T1f, (((0,), (0,)), ((), ())),
                         preferred_element_type=jnp.float32)
    return g1_blk + ea + c1


def _stats1_body(g1_ref, eat_ref, T1_ref, c1_ref, st_ref):
    i = pl.program_id(0)

    @pl.when(i == 0)
    def _():
        st_ref[...] = jnp.zeros((2, H), jnp.float32)
    z1 = _edge_z1(g1_ref[...], eat_ref[...], T1_ref[...], c1_ref[...])
    rows = i * BLK + lax.broadcasted_iota(jnp.int32, (BLK, 1), 0)
    zm = jnp.where(rows < E, z1, 0.0)
    s = jnp.sum(zm, axis=0)
    sq = jnp.sum(zm * zm, axis=0)
    st_ref[...] = st_ref[...] + jnp.stack([s, sq])


def _tc_stats1(G1, eaT, T1f, c1):
    return pl.pallas_call(
        _stats1_body,
        grid=(EB,),
        in_specs=[
            pl.BlockSpec((BLK, H), lambda i: (i, 0)),
            pl.BlockSpec((3, BLK), lambda i: (0, i)),
            pl.BlockSpec((3, H), lambda i: (0, 0)),
            pl.BlockSpec((1, H), lambda i: (0, 0)),
        ],
        out_specs=pl.BlockSpec((2, H), lambda i: (0, 0)),
        out_shape=jax.ShapeDtypeStruct((2, H), jnp.float32),
    )(G1, eaT, T1f, c1)


def _bn_affine(st, g, b):
    m = st[0, :] / float(E)
    var = st[1, :] / float(E) - m * m
    a = g * lax.rsqrt(var + 1e-5)
    return a, b - m * a


def _z2_body(g1_ref, eat_ref, T1_ref, c1_ref, st1_ref, g1g_ref, g1b_ref,
             g2_ref, M2_ref, c2_ref, z2_ref, st2_ref):
    i = pl.program_id(0)

    @pl.when(i == 0)
    def _():
        st2_ref[...] = jnp.zeros((2, H), jnp.float32)
    z1 = _edge_z1(g1_ref[...], eat_ref[...], T1_ref[...], c1_ref[...])
    a, b = _bn_affine(st1_ref[...], g1g_ref[...], g1b_ref[...])
    ef1 = jnp.maximum(z1 * a[None, :] + b[None, :], 0.0)
    z2 = (jnp.dot(ef1, M2_ref[...], preferred_element_type=jnp.float32)
          + g2_ref[...] + c2_ref[...])
    z2_ref[...] = z2
    rows = i * BLK + lax.broadcasted_iota(jnp.int32, (BLK, 1), 0)
    zm = jnp.where(rows < E, z2, 0.0)
    st2_ref[...] = st2_ref[...] + jnp.stack(
        [jnp.sum(zm, axis=0), jnp.sum(zm * zm, axis=0)])


def _tc_z2(G1, eaT, T1f, c1, st1, e1bng, e1bnb, G2, M2, c2):
    return pl.pallas_call(
        _z2_body,
        grid=(EB,),
        in_specs=[
            pl.BlockSpec((BLK, H), lambda i: (i, 0)),
            pl.BlockSpec((3, BLK), lambda i: (0, i)),
            pl.BlockSpec((3, H), lambda i: (0, 0)),
            pl.BlockSpec((1, H), lambda i: (0, 0)),
            pl.BlockSpec((2, H), lambda i: (0, 0)),
            pl.BlockSpec((H,), lambda i: (0,)),
            pl.BlockSpec((H,), lambda i: (0,)),
            pl.BlockSpec((BLK, H), lambda i: (i, 0)),
            pl.BlockSpec((H, H), lambda i: (0, 0)),
            pl.BlockSpec((1, H), lambda i: (0, 0)),
        ],
        out_specs=(pl.BlockSpec((BLK, H), lambda i: (i, 0)),
                   pl.BlockSpec((2, H), lambda i: (0, 0))),
        out_shape=(jax.ShapeDtypeStruct((EP, H), jnp.float32),
                   jax.ShapeDtypeStruct((2, H), jnp.float32)),
    )(G1, eaT, T1f, c1, st1, e1bng, e1bnb, G2, M2, c2)


def _scores_body(z2_ref, st2_ref, g2g_ref, g2b_ref, pwT_ref, pb_ref,
                 sc_ref):
    a, b = _bn_affine(st2_ref[...], g2g_ref[...], g2b_ref[...])
    ef2 = jnp.maximum(z2_ref[...] * a[None, :] + b[None, :], 0.0)
    s = jnp.sum(ef2 * pwT_ref[0, :][None, :], axis=1) + pb_ref[0, 0]
    sc_ref[...] = s.reshape(BLK // 128, 128)


def _tc_scores(z2, st2, e2bng, e2bnb, polWT, polb):
    return pl.pallas_call(
        _scores_body,
        grid=(EB,),
        in_specs=[
            pl.BlockSpec((BLK, H), lambda i: (i, 0)),
            pl.BlockSpec((2, H), lambda i: (0, 0)),
            pl.BlockSpec((H,), lambda i: (0,)),
            pl.BlockSpec((H,), lambda i: (0,)),
            pl.BlockSpec((1, H), lambda i: (0, 0)),
            pl.BlockSpec(memory_space=pltpu.SMEM),
        ],
        out_specs=pl.BlockSpec((BLK // 128, 128), lambda i: (i, 0)),
        out_shape=jax.ShapeDtypeStruct((EPR, 128), jnp.float32),
    )(z2, st2, e2bng, e2bnb, polWT, polb)


def _max_body(sc_ref, src_ref, dst_ref, m_ref):
    i = pl.program_id(0)

    @pl.when(i == 0)
    def _():
        m_ref[0, 0] = 0.0
    keep = src_ref[...] < dst_ref[...]
    m = jnp.max(jnp.where(keep, sc_ref[...], 0.0))
    m_ref[0, 0] = jnp.maximum(m_ref[0, 0], m)


def _denom_body(sc_ref, src_ref, dst_ref, m_ref, d_ref):
    i = pl.program_id(0)

    @pl.when(i == 0)
    def _():
        d_ref[0, 0] = 0.0
    mx = m_ref[0, 0]
    em = jnp.exp(-mx)
    keep = src_ref[...] < dst_ref[...]
    safe = jnp.where(keep, sc_ref[...], mx)
    term = jnp.where(keep, jnp.exp(safe - mx) - em, 0.0)
    d_ref[0, 0] = d_ref[0, 0] + jnp.sum(term)


def _vals_body(n_ref, m_ref, d_ref, sc_ref, src_ref, dst_ref,
               val_ref, idx_ref, base_ref):
    nn = n_ref[0, 0]
    mx = m_ref[0, 0]
    em = jnp.exp(-mx)
    denom = float(NUM_POLICY) * em + d_ref[0, 0]
    base = em / denom
    base_ref[0, 0] = base
    src = src_ref[...]
    dst = dst_ref[...]
    keep = src < dst
    safe = jnp.where(keep, sc_ref[...], mx)
    val_ref[...] = jnp.where(keep, jnp.exp(safe - mx) / denom, base)
    tri = src * (2 * nn - src - 1) // 2 + (dst - src - 1)
    idx_ref[...] = jnp.where(keep, tri, NUM_POLICY - 1)


def _tc_policy_vals(n, scores2d, src2d, dst2d):
    sblk = [
        pl.BlockSpec((EPR // SB, 128), lambda i: (i, 0)),
        pl.BlockSpec((EPR // SB, 128), lambda i: (i, 0)),
        pl.BlockSpec((EPR // SB, 128), lambda i: (i, 0)),
    ]
    smem = pl.BlockSpec(memory_space=pltpu.SMEM)
    m = pl.pallas_call(
        _max_body, grid=(SB,), in_specs=sblk, out_specs=smem,
        out_shape=jax.ShapeDtypeStruct((1, 1), jnp.float32),
    )(scores2d, src2d, dst2d)
    d = pl.pallas_call(
        _denom_body, grid=(SB,), in_specs=sblk + [smem], out_specs=smem,
        out_shape=jax.ShapeDtypeStruct((1, 1), jnp.float32),
    )(scores2d, src2d, dst2d, m)
    return pl.pallas_call(
        _vals_body, grid=(SB,), in_specs=[smem, smem, smem] + sblk,
        out_specs=(pl.BlockSpec((EPR // SB, 128), lambda i: (i, 0)),
                   pl.BlockSpec((EPR // SB, 128), lambda i: (i, 0)),
                   smem),
        out_shape=(jax.ShapeDtypeStruct((EPR, 128), jnp.float32),
                   jax.ShapeDtypeStruct((EPR, 128), jnp.int32),
                   jax.ShapeDtypeStruct((1, 1), jnp.float32)),
    )(n, m, d, scores2d, src2d, dst2d)


def _fill_body(b_ref, out_ref):
    out_ref[...] = jnp.full((FILL_BLK,), b_ref[0, 0], jnp.float32)


def _tc_fill(base):
    return pl.pallas_call(
        _fill_body,
        grid=(pl.cdiv(NUM_POLICY, FILL_BLK),),
        in_specs=[pl.BlockSpec(memory_space=pltpu.SMEM)],
        out_specs=pl.BlockSpec((FILL_BLK,), lambda i: (i,)),
        out_shape=jax.ShapeDtypeStruct((NUM_POLICY,), jnp.float32),
    )(base)


def kernel(edge_index, edge_attr, emb, eeW, eeb, g1W, g1b, bn1g, bn1b,
           g2W, g2b, bn2g, bn2b, e1epW, e1epb, e1npW, e1npb, e1cbW, e1cbb,
           e1bng, e1bnb, e2epW, e2epb, e2npW, e2npb, e2cbW, e2cbb, e2bng,
           e2bnb, polW, polb, v1W, v1b, v2W, v2b):
    ei = edge_index.astype(jnp.int32)
    ei_pad = jnp.pad(ei, ((0, 0), (0, EP - E)), constant_values=V)
    src2d = ei_pad[0].reshape(EPR, 128)
    dst2d = ei_pad[1].reshape(EPR, 128)
    src64 = ei_pad[0].reshape(EP // GSZ, GSZ)
    dst64 = ei_pad[1].reshape(EP // GSZ, GSZ)
    eaT = jnp.pad(edge_attr.T, ((0, 0), (0, EP - E)))

    n = _tc_n(ei.reshape(2500, 128))
    S1, D1, h1, T1f, c1, M2, c2 = _tc_prep1(
        n, emb, g1W, e1npW, e1cbW, eeW, eeb, e1epW, e1epb, e1npb, e1cbb,
        e2epW, e2epb, e2npb, e2cbW, e2cbb)
    cnt = _sc_degree(dst2d)
    dis, hd1 = _tc_dis(cnt, h1)
    G1, parts1 = _sc_layer(S1, D1, hd1, src64, dst64)
    S2, D2, hd2 = _tc_layer2(n, parts1, hd1, dis, g1b, bn1g, bn1b, g2W,
                             e2npW, e2cbW)
    G2, parts2 = _sc_layer(S2, D2, hd2, src64, dst64)
    value = _tc_value(n, parts2, hd2, dis, g2b, bn2g, bn2b, v1W, v1b,
                      v2W, v2b)

    st1 = _tc_stats1(G1, eaT, T1f, c1)
    z2, st2 = _tc_z2(G1, eaT, T1f, c1, st1, e1bng, e1bnb, G2, M2, c2)
    scores2d = _tc_scores(z2, st2, e2bng, e2bnb, polW.T,
                          polb.reshape(1, 1))
    vals2d, idx2d, base = _tc_policy_vals(n, scores2d, src2d, dst2d)

    pol = _tc_fill(base)
    pol_ref = jax.new_ref(pol)
    _sc_policy_scatter(pol_ref, idx2d, vals2d)
    policy = jax.freeze(pol_ref)
    return policy, value


# speedup vs baseline: 3.0316x; 3.0316x over previous
import functools

import numpy as np

import jax
import jax.numpy as jnp
from jax import lax
from jax.experimental import pallas as pl
from jax.experimental.pallas import tpu as pltpu
from jax.experimental.pallas import tpu_sc as plsc

V = 10000
H = 128
E = 160000
NUM_POLICY = V * (V - 1) // 2

NC = 2
NS = 16
NWK = NC * NS
L = 16

VP = 10240
SLAB = VP // NS
EP = 163840
EPW = EP // NWK
GW = EPW // 128
EPR = EP // 128
BLK = 2048
EB = EP // BLK
SB = 10
FILL_BLK = 1048576

_sc_mesh = functools.partial(
    plsc.VectorSubcoreMesh, core_axis_name="c", subcore_axis_name="s")


def _wid():
    return lax.axis_index("c") * NS + lax.axis_index("s")


def _sc_degree_body(dst2d, cnt, acc, zbuf, ones, idxv, sem):
    c = lax.axis_index("c")
    s = lax.axis_index("s")
    w = _wid()

    def zero16(k, _):
        zbuf[pl.ds(k * L, L)] = jnp.zeros((L,), jnp.float32)
        return _
    lax.fori_loop(0, SLAB // L, zero16, None)

    def one16(k, _):
        ones[pl.ds(k * L, L)] = jnp.full((L,), 1.0, jnp.float32)
        return _
    lax.fori_loop(0, 128 // L, one16, None)

    pltpu.sync_copy(zbuf, acc.at[pl.ds(s * SLAB, SLAB)])
    plsc.subcore_barrier()

    pltpu.sync_copy(dst2d.at[pl.ds(w * GW, GW)], idxv)

    def body(j, _):
        pltpu.sync_copy(ones, acc.at[idxv.at[j]], add=True)
        return _
    lax.fori_loop(0, GW, body, None)

    plsc.subcore_barrier()
    pltpu.sync_copy(acc.at[pl.ds(s * SLAB, SLAB)],
                    cnt.at[c, pl.ds(s * SLAB, SLAB)])


def _sc_degree(dst2d):
    return pl.kernel(
        _sc_degree_body,
        out_type=jax.ShapeDtypeStruct((NC, VP), jnp.float32),
        mesh=_sc_mesh(),
        scratch_types=[
            pltpu.VMEM_SHARED((VP,), jnp.float32),
            pltpu.VMEM((SLAB,), jnp.float32),
            pltpu.VMEM((128,), jnp.float32),
            pltpu.VMEM((GW, 128), jnp.int32),
            pltpu.SemaphoreType.DMA,
        ],
    )(dst2d)


GSZ = 64
GW2 = EPW // GSZ


def _sc_layer_body(tabS, tabD, hd, src2d, dst2d, gout, parts,
                   acc, idxs, idxd, bufA, bufB, bufC, semA, semB, semC):
    c = lax.axis_index("c")
    s = lax.axis_index("s")
    w = _wid()

    def zero16(t, _):
        row = t // 8
        col = (t % 8) * L
        bufC[row, pl.ds(col, L)] = jnp.zeros((L,), jnp.float32)
        return _
    lax.fori_loop(0, GSZ * 8, zero16, None)

    def zslab(k, _):
        pltpu.sync_copy(bufC, acc.at[pl.ds(s * SLAB + k * GSZ, GSZ)])
        return _
    lax.fori_loop(0, SLAB // GSZ, zslab, None)
    plsc.subcore_barrier()

    pltpu.sync_copy(src2d.at[pl.ds(w * GW2, GW2)], idxs)
    pltpu.sync_copy(dst2d.at[pl.ds(w * GW2, GW2)], idxd)

    def body(j, _):
        da = pltpu.async_copy(tabS.at[idxs.at[j]], bufA, semA)
        db = pltpu.async_copy(tabD.at[idxd.at[j]], bufB, semB)
        dc = pltpu.async_copy(hd.at[idxs.at[j]], bufC, semC)
        da.wait()
        db.wait()

        @plsc.parallel_loop(0, GSZ * 8, 1, unroll=8)
        def add16(t):
            row = t // 8
            col = (t % 8) * L
            bufA[row, pl.ds(col, L)] = (bufA[row, pl.ds(col, L)]
                                        + bufB[row, pl.ds(col, L)])

        pltpu.sync_copy(bufA, gout.at[pl.ds(w * EPW + j * GSZ, GSZ)])
        dc.wait()
        pltpu.sync_copy(bufC, acc.at[idxd.at[j]], add=True)
        return _
    lax.fori_loop(0, GW2, body, None)

    plsc.subcore_barrier()
    pltpu.sync_copy(acc.at[pl.ds(s * SLAB, SLAB)],
                    parts.at[c, pl.ds(s * SLAB, SLAB)])


def _sc_layer(tabS, tabD, hd, src64, dst64):
    return pl.kernel(
        _sc_layer_body,
        out_type=(jax.ShapeDtypeStruct((EP, H), jnp.float32),
                  jax.ShapeDtypeStruct((NC, VP, H), jnp.float32)),
        mesh=_sc_mesh(),
        scratch_types=[
            pltpu.VMEM_SHARED((VP, H), jnp.float32),
            pltpu.VMEM((GW2, GSZ), jnp.int32),
            pltpu.VMEM((GW2, GSZ), jnp.int32),
            pltpu.VMEM((GSZ, H), jnp.float32),
            pltpu.VMEM((GSZ, H), jnp.float32),
            pltpu.VMEM((GSZ, H), jnp.float32),
            pltpu.SemaphoreType.DMA,
            pltpu.SemaphoreType.DMA,
            pltpu.SemaphoreType.DMA,
        ],
    )(tabS, tabD, hd, src64, dst64)


CHN = 1310720
RSZ = CHN + 16
SLC = CHN // NS
SCLO1 = 24997504
NCHP = 20
ROWS_W = EPR // NS


def _sc_policy_body(idx2d, dval2d, base2d, bounds, pol,
                    region, idxbuf, dvalbuf, ilrow, basebuf, bbuf, bb2,
                    bounce, sem):
    c = lax.axis_index("c")
    s = lax.axis_index("s")
    lo_cs = c * SCLO1
    hi_cs = SCLO1 + c * (NUM_POLICY - SCLO1)

    pltpu.sync_copy(bounds.at[c], bbuf)
    lo_cv = bbuf[0, pl.ds(0, L)]
    hi_cv = bbuf[1, pl.ds(0, L)]

    pltpu.sync_copy(base2d, bb2)
    base = bb2[0, pl.ds(0, L)]

    def bfill(k, _):
        basebuf[pl.ds(k * L, L)] = base
        return _
    lax.fori_loop(0, 2048 // L, bfill, None)

    pltpu.sync_copy(idx2d.at[pl.ds(s * ROWS_W, ROWS_W)], idxbuf)
    pltpu.sync_copy(dval2d.at[pl.ds(s * ROWS_W, ROWS_W)], dvalbuf)

    zero_v = jnp.full((L,), 0, jnp.int32)
    eight_v = jnp.full((L,), 8, jnp.int32)
    chn_v = jnp.full((L,), CHN, jnp.int32)
    chn8_v = jnp.full((L,), CHN + 8, jnp.int32)

    def per_chunk(k, lo_kv):
        lo_ks = jnp.minimum(lo_cs + k * CHN, hi_cs - CHN)
        lo_kv = jnp.minimum(lo_kv, hi_cv - chn_v)

        def refill(t, _):
            pltpu.sync_copy(
                basebuf, region.at[pl.ds(8 + s * SLC + t * 2048, 2048)])
            return _
        lax.fori_loop(0, SLC // 2048, refill, None)
        plsc.subcore_barrier()

        def group(g, _):
            def lane(t, _):
                col = t * L
                iv = idxbuf[g, pl.ds(col, L)]
                il = jnp.minimum(
                    jnp.maximum(iv - lo_kv + eight_v, zero_v), chn8_v)
                ilrow[0, pl.ds(col, L)] = il
                return _
            lax.fori_loop(0, 8, lane, None)
            pltpu.sync_copy(dvalbuf.at[g], region.at[ilrow.at[0]],
                            add=True)
            return _
        lax.fori_loop(0, ROWS_W, group, None)
        plsc.subcore_barrier()

        def drain(t, _):
            pltpu.sync_copy(
                region.at[pl.ds(8 + s * SLC + t * 8192, 8192)], bounce)
            pltpu.sync_copy(
                bounce, pol.at[pl.ds(lo_ks + s * SLC + t * 8192, 8192)])
            return _
        lax.fori_loop(0, SLC // 8192, drain, None)
        plsc.subcore_barrier()
        return lo_kv + chn_v
    lax.fori_loop(0, NCHP, per_chunk, lo_cv)


def _sc_policy(idx2d, dval2d, base2d, bounds):
    return pl.kernel(
        _sc_policy_body,
        out_type=jax.ShapeDtypeStruct((NUM_POLICY,), jnp.float32),
        mesh=_sc_mesh(),
        scratch_types=[
            pltpu.VMEM_SHARED((RSZ,), jnp.float32),
            pltpu.VMEM((ROWS_W, 128), jnp.int32),
            pltpu.VMEM((ROWS_W, 128), jnp.float32),
            pltpu.VMEM((1, 128), jnp.int32),
            pltpu.VMEM((2048,), jnp.float32),
            pltpu.VMEM((2, L), jnp.int32),
            pltpu.VMEM((8, L), jnp.float32),
            pltpu.VMEM((8192,), jnp.float32),
            pltpu.SemaphoreType.DMA,
        ],
    )(idx2d, dval2d, base2d, bounds)


def _full(a):
    nd = a.ndim
    return pl.BlockSpec(a.shape, lambda *_, _nd=nd: (0,) * _nd)


def _n_body(ei_ref, n_ref):
    i = pl.program_id(0)

    @pl.when(i == 0)
    def _():
        n_ref[0, 0] = 0
    m = jnp.max(ei_ref[...])
    n_ref[0, 0] = jnp.maximum(n_ref[0, 0], m + 1)


def _tc_n(ei_flat2d):
    return pl.pallas_call(
        _n_body,
        grid=(1,),
        in_specs=[pl.BlockSpec((2500, 128), lambda i: (0, 0))],
        out_specs=pl.BlockSpec(memory_space=pltpu.SMEM),
        out_shape=jax.ShapeDtypeStruct((1, 1), jnp.int32),
    )(ei_flat2d)


def _prep1_body(n_ref, emb, g1W, e1npW, e1cbW, eeW, eeb, e1epW, e1epb,
                e1npb, e1cbb, e2epW, e2epb, e2npb, e2cbW, e2cbb,
                S1, D1, h1, T1f, c1o, M2o, c2o):
    nn = n_ref[0, 0]
    rows = lax.broadcasted_iota(jnp.int32, (V, 1), 0)
    x0 = jnp.where(rows < nn, emb[...], 0.0)
    dot = functools.partial(jnp.dot, preferred_element_type=jnp.float32)
    pad = jnp.zeros((VP - V, H), jnp.float32)

    S1[pl.ds(0, V), :] = dot(x0, dot(e1npW[0:H, :], e1cbW[0:H, :]))
    S1[pl.ds(V, VP - V), :] = pad
    D1[pl.ds(0, V), :] = dot(x0, dot(e1npW[H:2 * H, :], e1cbW[0:H, :]))
    D1[pl.ds(V, VP - V), :] = pad
    h1[pl.ds(0, V), :] = dot(x0, g1W[...])
    h1[pl.ds(V, VP - V), :] = pad

    cbb = e1cbW[H:2 * H, :]
    T1f[...] = dot(eeW[...], dot(e1epW[...], cbb))
    c1o[...] = (dot(e1npb[...].reshape(1, H), e1cbW[0:H, :])
                + dot((dot(eeb[...].reshape(1, H), e1epW[...])
                       + e1epb[...].reshape(1, H)), cbb)
                + e1cbb[...].reshape(1, H))
    M2o[...] = dot(e2epW[...], e2cbW[H:2 * H, :])
    c2o[...] = (dot(e2npb[...].reshape(1, H), e2cbW[0:H, :])
                + dot(e2epb[...].reshape(1, H), e2cbW[H:2 * H, :])
                + e2cbb[...].reshape(1, H))


def _tc_prep1(n, emb, g1W, e1npW, e1cbW, eeW, eeb, e1epW, e1epb, e1npb,
              e1cbb, e2epW, e2epb, e2npb, e2cbW, e2cbb):
    return pl.pallas_call(
        _prep1_body,
        in_specs=[pl.BlockSpec(memory_space=pltpu.SMEM)] + [
            _full(a)
            for a in (emb, g1W, e1npW, e1cbW, eeW, eeb, e1epW, e1epb,
                      e1npb, e1cbb, e2epW, e2epb, e2npb, e2cbW, e2cbb)],
        out_shape=(
            jax.ShapeDtypeStruct((VP, H), jnp.float32),
            jax.ShapeDtypeStruct((VP, H), jnp.float32),
            jax.ShapeDtypeStruct((VP, H), jnp.float32),
            jax.ShapeDtypeStruct((3, H), jnp.float32),
            jax.ShapeDtypeStruct((1, H), jnp.float32),
            jax.ShapeDtypeStruct((H, H), jnp.float32),
            jax.ShapeDtypeStruct((1, H), jnp.float32),
        ),
    )(n, emb, g1W, e1npW, e1cbW, eeW, eeb, e1epW, e1epb, e1npb, e1cbb,
      e2epW, e2epb, e2npb, e2cbW, e2cbb)


def _dis_body(cnt, h1, dis, hd1):
    c = cnt[0, :] + cnt[1, :] + 1.0
    d = lax.rsqrt(c).reshape(VP, 1)
    dis[...] = d
    hd1[...] = h1[...] * d


def _tc_dis(cnt, h1):
    return pl.pallas_call(
        _dis_body,
        out_shape=(jax.ShapeDtypeStruct((VP, 1), jnp.float32),
                   jax.ShapeDtypeStruct((VP, H), jnp.float32)),
    )(cnt, h1)


def _node_finish(parts, hd, dis, b, bng, bnb, nn):
    acc = parts[0] + parts[1]
    out = dis * (acc + hd) + b[None, :]
    rows = lax.broadcasted_iota(jnp.int32, (VP, 1), 0)
    mask = rows < nn
    nf = nn.astype(jnp.float32)
    m = jnp.sum(jnp.where(mask, out, 0.0), axis=0, keepdims=True) / nf
    cen = jnp.where(mask, out - m, 0.0)
    var = jnp.sum(cen * cen, axis=0, keepdims=True) / nf
    y = bng[None, :] * (out - m) * lax.rsqrt(var + 1e-5) + bnb[None, :]
    return jnp.where(mask, jnp.maximum(y, 0.0), 0.0)


def _layer2_body(n_ref, parts1, hd1, dis, g1b, bn1g, bn1b, g2W, e2npW,
                 e2cbW, S2, D2, hd2):
    nn = n_ref[0, 0]
    x1 = _node_finish(parts1[...], hd1[...], dis[...], g1b[...],
                      bn1g[...], bn1b[...], nn)
    dot = functools.partial(jnp.dot, preferred_element_type=jnp.float32)
    S2[...] = dot(x1, dot(e2npW[0:H, :], e2cbW[0:H, :]))
    D2[...] = dot(x1, dot(e2npW[H:2 * H, :], e2cbW[0:H, :]))
    hd2[...] = dot(x1, g2W[...]) * dis[...]


def _tc_layer2(n, parts1, hd1, dis, g1b, bn1g, bn1b, g2W, e2npW, e2cbW):
    return pl.pallas_call(
        _layer2_body,
        in_specs=[pl.BlockSpec(memory_space=pltpu.SMEM)] + [
            _full(a)
            for a in (parts1, hd1, dis, g1b, bn1g, bn1b, g2W, e2npW,
                      e2cbW)],
        out_shape=(jax.ShapeDtypeStruct((VP, H), jnp.float32),
                   jax.ShapeDtypeStruct((VP, H), jnp.float32),
                   jax.ShapeDtypeStruct((VP, H), jnp.float32)),
    )(n, parts1, hd1, dis, g1b, bn1g, bn1b, g2W, e2npW, e2cbW)


def _value_body(n_ref, parts2, hd2, dis, g2b, bn2g, bn2b, v1W, v1b, v2W,
                v2b, val):
    nn = n_ref[0, 0]
    x2 = _node_finish(parts2[...], hd2[...], dis[...], g2b[...],
                      bn2g[...], bn2b[...], nn)
    nf = nn.astype(jnp.float32)
    xmean = jnp.sum(x2, axis=0, keepdims=True) / nf
    dot = functools.partial(jnp.dot, preferred_element_type=jnp.float32)
    t = jnp.maximum(dot(xmean, v1W[...]) + v1b[...][None, :], 0.0)
    val[...] = jnp.tanh(dot(t, v2W[...]) + v2b[...][None, :])


def _tc_value(n, parts2, hd2, dis, g2b, bn2g, bn2b, v1W, v1b, v2W, v2b):
    return pl.pallas_call(
        _value_body,
        in_specs=[pl.BlockSpec(memory_space=pltpu.SMEM)] + [
            _full(a)
            for a in (parts2, hd2, dis, g2b, bn2g, bn2b, v1W, v1b, v2W,
                      v2b)],
        out_shape=jax.ShapeDtypeStruct((1, 1), jnp.float32),
    )(n, parts2, hd2, dis, g2b, bn2g, bn2b, v1W, v1b, v2W, v2b)


def _edge_z1(g1_blk, eat_blk, T1f, c1):
    ea = lax.dot_general(eat_blk, T1f, (((0,), (0,)), ((), ())),
                         preferred_element_type=jnp.float32)
    return g1_blk + ea + c1


def _stats1_body(g1_ref, eat_ref, T1_ref, c1_ref, st_ref):
    i = pl.program_id(0)

    @pl.when(i == 0)
    def _():
        st_ref[...] = jnp.zeros((2, H), jnp.float32)
    z1 = _edge_z1(g1_ref[...], eat_ref[...], T1_ref[...], c1_ref[...])
    rows = i * BLK + lax.broadcasted_iota(jnp.int32, (BLK, 1), 0)
    zm = jnp.where(rows < E, z1, 0.0)
    s = jnp.sum(zm, axis=0)
    sq = jnp.sum(zm * zm, axis=0)
    st_ref[...] = st_ref[...] + jnp.stack([s, sq])


def _tc_stats1(G1, eaT, T1f, c1):
    return pl.pallas_call(
        _stats1_body,
        grid=(EB,),
        in_specs=[
            pl.BlockSpec((BLK, H), lambda i: (i, 0)),
            pl.BlockSpec((3, BLK), lambda i: (0, i)),
            pl.BlockSpec((3, H), lambda i: (0, 0)),
            pl.BlockSpec((1, H), lambda i: (0, 0)),
        ],
        out_specs=pl.BlockSpec((2, H), lambda i: (0, 0)),
        out_shape=jax.ShapeDtypeStruct((2, H), jnp.float32),
    )(G1, eaT, T1f, c1)


def _bn_affine(st, g, b):
    m = st[0, :] / float(E)
    var = st[1, :] / float(E) - m * m
    a = g * lax.rsqrt(var + 1e-5)
    return a, b - m * a


def _z2_body(g1_ref, eat_ref, T1_ref, c1_ref, st1_ref, g1g_ref, g1b_ref,
             g2_ref, M2_ref, c2_ref, z2_ref, st2_ref):
    i = pl.program_id(0)

    @pl.when(i == 0)
    def _():
        st2_ref[...] = jnp.zeros((2, H), jnp.float32)
    z1 = _edge_z1(g1_ref[...], eat_ref[...], T1_ref[...], c1_ref[...])
    a, b = _bn_affine(st1_ref[...], g1g_ref[...], g1b_ref[...])
    ef1 = jnp.maximum(z1 * a[None, :] + b[None, :], 0.0)
    z2 = (jnp.dot(ef1, M2_ref[...], preferred_element_type=jnp.float32)
          + g2_ref[...] + c2_ref[...])
    z2_ref[...] = z2
    rows = i * BLK + lax.broadcasted_iota(jnp.int32, (BLK, 1), 0)
    zm = jnp.where(rows < E, z2, 0.0)
    st2_ref[...] = st2_ref[...] + jnp.stack(
        [jnp.sum(zm, axis=0), jnp.sum(zm * zm, axis=0)])


def _tc_z2(G1, eaT, T1f, c1, st1, e1bng, e1bnb, G2, M2, c2):
    return pl.pallas_call(
        _z2_body,
        grid=(EB,),
        in_specs=[
            pl.BlockSpec((BLK, H), lambda i: (i, 0)),
            pl.BlockSpec((3, BLK), lambda i: (0, i)),
            pl.BlockSpec((3, H), lambda i: (0, 0)),
            pl.BlockSpec((1, H), lambda i: (0, 0)),
            pl.BlockSpec((2, H), lambda i: (0, 0)),
            pl.BlockSpec((H,), lambda i: (0,)),
            pl.BlockSpec((H,), lambda i: (0,)),
            pl.BlockSpec((BLK, H), lambda i: (i, 0)),
            pl.BlockSpec((H, H), lambda i: (0, 0)),
            pl.BlockSpec((1, H), lambda i: (0, 0)),
        ],
        out_specs=(pl.BlockSpec((BLK, H), lambda i: (i, 0)),
                   pl.BlockSpec((2, H), lambda i: (0, 0))),
        out_shape=(jax.ShapeDtypeStruct((EP, H), jnp.float32),
                   jax.ShapeDtypeStruct((2, H), jnp.float32)),
    )(G1, eaT, T1f, c1, st1, e1bng, e1bnb, G2, M2, c2)


def _scores_body(z2_ref, st2_ref, g2g_ref, g2b_ref, pwT_ref, pb_ref,
                 sc_ref):
    a, b = _bn_affine(st2_ref[...], g2g_ref[...], g2b_ref[...])
    ef2 = jnp.maximum(z2_ref[...] * a[None, :] + b[None, :], 0.0)
    s = jnp.sum(ef2 * pwT_ref[0, :][None, :], axis=1) + pb_ref[0, 0]
    sc_ref[...] = s.reshape(BLK // 128, 128)


def _tc_scores(z2, st2, e2bng, e2bnb, polWT, polb):
    return pl.pallas_call(
        _scores_body,
        grid=(EB,),
        in_specs=[
            pl.BlockSpec((BLK, H), lambda i: (i, 0)),
            pl.BlockSpec((2, H), lambda i: (0, 0)),
            pl.BlockSpec((H,), lambda i: (0,)),
            pl.BlockSpec((H,), lambda i: (0,)),
            pl.BlockSpec((1, H), lambda i: (0, 0)),
            pl.BlockSpec(memory_space=pltpu.SMEM),
        ],
        out_specs=pl.BlockSpec((BLK // 128, 128), lambda i: (i, 0)),
        out_shape=jax.ShapeDtypeStruct((EPR, 128), jnp.float32),
    )(z2, st2, e2bng, e2bnb, polWT, polb)


def _max_body(sc_ref, src_ref, dst_ref, m_ref):
    i = pl.program_id(0)

    @pl.when(i == 0)
    def _():
        m_ref[0, 0] = 0.0
    keep = src_ref[...] < dst_ref[...]
    m = jnp.max(jnp.where(keep, sc_ref[...], 0.0))
    m_ref[0, 0] = jnp.maximum(m_ref[0, 0], m)


def _denom_body(sc_ref, src_ref, dst_ref, m_ref, d_ref):
    i = pl.program_id(0)

    @pl.when(i == 0)
    def _():
        d_ref[0, 0] = 0.0
    mx = m_ref[0, 0]
    em = jnp.exp(-mx)
    keep = src_ref[...] < dst_ref[...]
    safe = jnp.where(keep, sc_ref[...], mx)
    term = jnp.where(keep, jnp.exp(safe - mx) - em, 0.0)
    d_ref[0, 0] = d_ref[0, 0] + jnp.sum(term)


def _vals_body(n_ref, m_ref, d_ref, sc_ref, src_ref, dst_ref,
               val_ref, idx_ref, base_ref):
    nn = n_ref[0, 0]
    mx = m_ref[0, 0]
    em = jnp.exp(-mx)
    denom = float(NUM_POLICY) * em + d_ref[0, 0]
    base = em / denom
    src = src_ref[...]
    dst = dst_ref[...]
    keep = src < dst
    safe = jnp.where(keep, sc_ref[...], mx)
    val_ref[...] = jnp.where(keep, jnp.exp(safe - mx) / denom - base, 0.0)
    tri = src * (2 * nn - src - 1) // 2 + (dst - src - 1)
    idx_ref[...] = jnp.where(keep, tri, NUM_POLICY)
    base_ref[...] = jnp.full((8, 16), base, jnp.float32)


def _tc_policy_vals(n, scores2d, src2d, dst2d):
    sblk = [
        pl.BlockSpec((EPR // SB, 128), lambda i: (i, 0)),
        pl.BlockSpec((EPR // SB, 128), lambda i: (i, 0)),
        pl.BlockSpec((EPR // SB, 128), lambda i: (i, 0)),
    ]
    smem = pl.BlockSpec(memory_space=pltpu.SMEM)
    m = pl.pallas_call(
        _max_body, grid=(SB,), in_specs=sblk, out_specs=smem,
        out_shape=jax.ShapeDtypeStruct((1, 1), jnp.float32),
    )(scores2d, src2d, dst2d)
    d = pl.pallas_call(
        _denom_body, grid=(SB,), in_specs=sblk + [smem], out_specs=smem,
        out_shape=jax.ShapeDtypeStruct((1, 1), jnp.float32),
    )(scores2d, src2d, dst2d, m)
    return pl.pallas_call(
        _vals_body, grid=(SB,), in_specs=[smem, smem, smem] + sblk,
        out_specs=(pl.BlockSpec((EPR // SB, 128), lambda i: (i, 0)),
                   pl.BlockSpec((EPR // SB, 128), lambda i: (i, 0)),
                   pl.BlockSpec((8, 16), lambda i: (0, 0))),
        out_shape=(jax.ShapeDtypeStruct((EPR, 128), jnp.float32),
                   jax.ShapeDtypeStruct((EPR, 128), jnp.int32),
                   jax.ShapeDtypeStruct((8, 16), jnp.float32)),
    )(n, m, d, scores2d, src2d, dst2d)


def kernel(edge_index, edge_attr, emb, eeW, eeb, g1W, g1b, bn1g, bn1b,
           g2W, g2b, bn2g, bn2b, e1epW, e1epb, e1npW, e1npb, e1cbW, e1cbb,
           e1bng, e1bnb, e2epW, e2epb, e2npW, e2npb, e2cbW, e2cbb, e2bng,
           e2bnb, polW, polb, v1W, v1b, v2W, v2b):
    ei = edge_index.astype(jnp.int32)
    ei_pad = jnp.pad(ei, ((0, 0), (0, EP - E)), constant_values=V)
    src2d = ei_pad[0].reshape(EPR, 128)
    dst2d = ei_pad[1].reshape(EPR, 128)
    src64 = ei_pad[0].reshape(EP // GSZ, GSZ)
    dst64 = ei_pad[1].reshape(EP // GSZ, GSZ)
    eaT = jnp.pad(edge_attr.T, ((0, 0), (0, EP - E)))

    n = _tc_n(ei.reshape(2500, 128))
    S1, D1, h1, T1f, c1, M2, c2 = _tc_prep1(
        n, emb, g1W, e1npW, e1cbW, eeW, eeb, e1epW, e1epb, e1npb, e1cbb,
        e2epW, e2epb, e2npb, e2cbW, e2cbb)
    cnt = _sc_degree(dst2d)
    dis, hd1 = _tc_dis(cnt, h1)
    G1, parts1 = _sc_layer(S1, D1, hd1, src64, dst64)
    S2, D2, hd2 = _tc_layer2(n, parts1, hd1, dis, g1b, bn1g, bn1b, g2W,
                             e2npW, e2cbW)
    G2, parts2 = _sc_layer(S2, D2, hd2, src64, dst64)
    value = _tc_value(n, parts2, hd2, dis, g2b, bn2g, bn2b, v1W, v1b,
                      v2W, v2b)

    st1 = _tc_stats1(G1, eaT, T1f, c1)
    z2, st2 = _tc_z2(G1, eaT, T1f, c1, st1, e1bng, e1bnb, G2, M2, c2)
    scores2d = _tc_scores(z2, st2, e2bng, e2bnb, polW.T,
                          polb.reshape(1, 1))
    dval2d, idx2d, baseflat = _tc_policy_vals(n, scores2d, src2d, dst2d)
    lows = np.array([0, SCLO1], dtype=np.int32)
    highs = np.array([SCLO1, NUM_POLICY], dtype=np.int32)
    bounds = jnp.asarray(
        np.stack([np.repeat(lows[:, None], L, 1),
                  np.repeat(highs[:, None], L, 1)], axis=1))
    policy = _sc_policy(idx2d, dval2d, baseflat, bounds)
    return policy, value
